# baseline (speedup 1.0000x reference)
import jax
import jax.numpy as jnp
from jax.experimental import pallas as pl
from jax.experimental.pallas import tpu as pltpu

B, N = 4, 1024
DIM, POS_DIM, K = 256, 16, 16
ALPHA, SIGMA = 0.1, 1.0


def _fused_kernel(states_ref, positions_ref, wf_ref, bf_ref, wv_ref, bv_ref,
                  ctx_ref, newpos_ref, flow_ref, dsel_ref):
    values = []
    for b in range(B):
        states = states_ref[b]
        positions = positions_ref[b]

        flow = jax.lax.dot_general(
            states, wf_ref[...],
            (((1,), (1,)), ((), ())),
            preferred_element_type=jnp.float32) + bf_ref[...][None, :]
        newpos = positions + ALPHA * flow
        flow_ref[b] = flow
        newpos_ref[b] = newpos

        values.append(jax.lax.dot_general(
            states, wv_ref[...],
            (((1,), (1,)), ((), ())),
            preferred_element_type=jnp.float32) + bv_ref[...][None, :])

        gram = jax.lax.dot_general(
            newpos, newpos,
            (((1,), (1,)), ((), ())),
            precision=jax.lax.Precision.HIGHEST,
            preferred_element_type=jnp.float32)
        sqn = jnp.sum(newpos * newpos, axis=1, keepdims=True)
        ones_row = jnp.ones((1, POS_DIM), dtype=jnp.float32)
        sqn_cols = jax.lax.dot_general(
            ones_row, newpos * newpos,
            (((1,), (1,)), ((), ())),
            precision=jax.lax.Precision.HIGHEST,
            preferred_element_type=jnp.float32)
        dsel_ref[:, b * N:(b + 1) * N] = jnp.maximum(
            sqn + sqn_cols - 2.0 * gram, 0.0)

    def body(_, m_prev):
        dsq = dsel_ref[...]
        return jnp.min(jnp.where(dsq > m_prev, dsq, jnp.float32(3e38)),
                       axis=0, keepdims=True)

    t = jax.lax.fori_loop(
        0, K + 1, body, jnp.full((1, B * N), -1.0, dtype=jnp.float32))

    iota_j = jax.lax.broadcasted_iota(jnp.int32, (N, N), 1)
    iota_i = jax.lax.broadcasted_iota(jnp.int32, (N, N), 0)
    ones_col = jnp.ones((N, 1), dtype=jnp.float32)
    for b in range(B):
        sq = dsel_ref[:, b * N:(b + 1) * N]
        d = jnp.sqrt(sq)
        wT = jnp.where((sq <= t[:, b * N:(b + 1) * N]) & (iota_j != iota_i),
                       jnp.exp(d * (-1.0 / (2.0 * SIGMA ** 2))), 0.0)
        s = jax.lax.dot_general(
            wT, ones_col,
            (((0,), (0,)), ((), ())),
            preferred_element_type=jnp.float32) + 1e-8
        ctx_ref[b] = jax.lax.dot_general(
            wT, values[b],
            (((0,), (0,)), ((), ())),
            preferred_element_type=jnp.float32) / s


def kernel(states, positions, W_flow, b_flow, W_val, b_val):
    out_shapes = (
        jax.ShapeDtypeStruct((B, N, DIM), jnp.float32),
        jax.ShapeDtypeStruct((B, N, POS_DIM), jnp.float32),
        jax.ShapeDtypeStruct((B, N, POS_DIM), jnp.float32),
    )
    context, new_positions, flow_vectors = pl.pallas_call(
        _fused_kernel,
        out_shape=out_shapes,
        scratch_shapes=[
            pltpu.VMEM((N, B * N), jnp.float32),
        ],
    )(states, positions, W_flow, b_flow, W_val, b_val)
    return (context, new_positions, flow_vectors)


# speedup vs baseline: 1.2544x; 1.2544x over previous
import jax
import jax.numpy as jnp
from jax.experimental import pallas as pl
from jax.experimental.pallas import tpu as pltpu

B, N = 4, 1024
DIM, POS_DIM, K = 256, 16, 16
ALPHA, SIGMA = 0.1, 1.0


def _fused_kernel(states_ref, positions_ref, wf_ref, bf_ref, wv_ref, bv_ref,
                  ctx_ref, newpos_ref, flow_ref, dsel_ref):
    states = states_ref[0]
    positions = positions_ref[0]

    flow = jax.lax.dot_general(
        states, wf_ref[...],
        (((1,), (1,)), ((), ())),
        preferred_element_type=jnp.float32) + bf_ref[...][None, :]
    newpos = positions + ALPHA * flow
    flow_ref[0] = flow
    newpos_ref[0] = newpos

    values = jax.lax.dot_general(
        states, wv_ref[...],
        (((1,), (1,)), ((), ())),
        preferred_element_type=jnp.float32) + bv_ref[...][None, :]

    gram = jax.lax.dot_general(
        newpos, newpos,
        (((1,), (1,)), ((), ())),
        precision=jax.lax.Precision.HIGHEST,
        preferred_element_type=jnp.float32)
    sqn = jnp.sum(newpos * newpos, axis=1, keepdims=True)
    ones_row = jnp.ones((1, POS_DIM), dtype=jnp.float32)
    sqn_cols = jax.lax.dot_general(
        ones_row, newpos * newpos,
        (((1,), (1,)), ((), ())),
        precision=jax.lax.Precision.HIGHEST,
        preferred_element_type=jnp.float32)
    sq = jnp.maximum(sqn + sqn_cols - 2.0 * gram, 0.0)

    dsel_ref[...] = sq

    def body(_, m_prev):
        dsq = dsel_ref[...]
        return jnp.min(jnp.where(dsq > m_prev, dsq, jnp.float32(3e38)),
                       axis=0, keepdims=True)

    t = jax.lax.fori_loop(
        0, K + 1, body, jnp.full((1, N), -1.0, dtype=jnp.float32))

    iota_j = jax.lax.broadcasted_iota(jnp.int32, (N, N), 1)
    iota_i = jax.lax.broadcasted_iota(jnp.int32, (N, N), 0)
    d = jnp.sqrt(sq)
    wT = jnp.where((sq <= t) & (iota_j != iota_i),
                   jnp.exp(d * (-1.0 / (2.0 * SIGMA ** 2))), 0.0)
    s = jax.lax.dot_general(
        wT, jnp.ones((N, 1), dtype=jnp.float32),
        (((0,), (0,)), ((), ())),
        preferred_element_type=jnp.float32) + 1e-8
    ctx = jax.lax.dot_general(
        wT, values,
        (((0,), (0,)), ((), ())),
        preferred_element_type=jnp.float32) / s
    ctx_ref[0] = ctx


def kernel(states, positions, W_flow, b_flow, W_val, b_val):
    grid = (B,)
    out_shapes = (
        jax.ShapeDtypeStruct((B, N, DIM), jnp.float32),
        jax.ShapeDtypeStruct((B, N, POS_DIM), jnp.float32),
        jax.ShapeDtypeStruct((B, N, POS_DIM), jnp.float32),
    )
    in_specs = [
        pl.BlockSpec((1, N, DIM), lambda b: (b, 0, 0)),
        pl.BlockSpec((1, N, POS_DIM), lambda b: (b, 0, 0)),
        pl.BlockSpec((POS_DIM, DIM), lambda b: (0, 0)),
        pl.BlockSpec((POS_DIM,), lambda b: (0,)),
        pl.BlockSpec((DIM, DIM), lambda b: (0, 0)),
        pl.BlockSpec((DIM,), lambda b: (0,)),
    ]
    out_specs = (
        pl.BlockSpec((1, N, DIM), lambda b: (b, 0, 0)),
        pl.BlockSpec((1, N, POS_DIM), lambda b: (b, 0, 0)),
        pl.BlockSpec((1, N, POS_DIM), lambda b: (b, 0, 0)),
    )
    context, new_positions, flow_vectors = pl.pallas_call(
        _fused_kernel,
        grid=grid,
        in_specs=in_specs,
        out_specs=out_specs,
        out_shape=out_shapes,
        scratch_shapes=[
            pltpu.VMEM((N, N), jnp.float32),
        ],
    )(states, positions, W_flow, b_flow, W_val, b_val)
    return (context, new_positions, flow_vectors)
